# baseline (speedup 1.0000x reference)
import jax
import jax.numpy as jnp
from jax.experimental import pallas as pl
from jax.experimental.pallas import tpu as pltpu


_BM = 200
_RING = 4


def _mm_kernel(f_hbm, a_hbm, o_hbm, f_vmem, f16, abuf, obuf, f_sem, a_sem, o_sem):
    k, d = f_vmem.shape
    n = a_hbm.shape[0] // _BM

    def a_copy(i):
        return pltpu.make_async_copy(
            a_hbm.at[pl.ds(i * _BM, _BM), :], abuf.at[i % _RING], a_sem.at[i % _RING])

    def o_copy(i):
        return pltpu.make_async_copy(
            obuf.at[i % 2], o_hbm.at[pl.ds(i * _BM, _BM), :], o_sem.at[i % 2])

    f_cp = pltpu.make_async_copy(f_hbm, f_vmem, f_sem)
    f_cp.start()
    for j in range(_RING):
        a_copy(j).start()
    f_cp.wait()
    f16[...] = f_vmem[...].astype(jnp.bfloat16)

    for i in range(n):
        a_copy(i).wait()
        if i >= 2:
            o_copy(i - 2).wait()
        obuf[i % 2, :, :] = jax.lax.dot_general(
            abuf[i % _RING], f16[...],
            (((1,), (0,)), ((), ())),
            precision=jax.lax.Precision.DEFAULT,
            preferred_element_type=jnp.float32,
        )
        o_copy(i).start()
        if i + _RING < n:
            a_copy(i + _RING).start()
    o_copy(n - 2).wait()
    o_copy(n - 1).wait()


def kernel(features, A):
    m, k = A.shape
    d = features.shape[1]
    return pl.pallas_call(
        _mm_kernel,
        in_specs=[
            pl.BlockSpec(memory_space=pltpu.MemorySpace.HBM),
            pl.BlockSpec(memory_space=pltpu.MemorySpace.HBM),
        ],
        out_specs=pl.BlockSpec(memory_space=pltpu.MemorySpace.HBM),
        out_shape=jax.ShapeDtypeStruct((m, d), jnp.float32),
        scratch_shapes=[
            pltpu.VMEM((k, d), jnp.float32),
            pltpu.VMEM((k, d), jnp.bfloat16),
            pltpu.VMEM((_RING, _BM, k), jnp.float32),
            pltpu.VMEM((2, _BM, d), jnp.float32),
            pltpu.SemaphoreType.DMA,
            pltpu.SemaphoreType.DMA((_RING,)),
            pltpu.SemaphoreType.DMA((2,)),
        ],
    )(features, A)
